# baseline (speedup 1.0000x reference)
import functools

import jax
import jax.numpy as jnp
from jax import lax
from jax.experimental import pallas as pl
from jax.experimental.pallas import tpu as pltpu
from jax.experimental.pallas import tpu_sc as plsc

_NC = 2
_NS = 16
_NW = _NC * _NS

_JBS = 100
_CJ = 8


@functools.cache
def _build(num_blocks, vocab, hidden, seq_len):
    del vocab
    blocks_per_w = num_blocks // _NW
    n_chunks = blocks_per_w // _CJ
    lanes = hidden // 16

    mesh = plsc.VectorSubcoreMesh(core_axis_name="c", subcore_axis_name="s")

    @functools.partial(
        pl.kernel,
        out_type=jax.ShapeDtypeStruct((num_blocks, _JBS, hidden), jnp.float32),
        mesh=mesh,
        compiler_params=pltpu.CompilerParams(use_tc_tiling_on_sc=False),
        scratch_types=[
            pltpu.VMEM((_CJ, _JBS), jnp.int32),
            pltpu.VMEM((_CJ, _JBS, hidden), jnp.float32),
            pltpu.VMEM((seq_len, hidden), jnp.float32),
            pltpu.SemaphoreType.DMA,
        ],
    )
    def emb(tok_hbm, table_hbm, pos_hbm, out_hbm, idx_v, rows_v, pos_v, sem):
        wid = lax.axis_index("s") * _NC + lax.axis_index("c")
        pltpu.sync_copy(pos_hbm, pos_v)

        def chunk(c, carry):
            jbase = wid * blocks_per_w + c * _CJ
            pltpu.sync_copy(tok_hbm.at[pl.ds(jbase, _CJ)], idx_v)
            copies = [
                pltpu.async_copy(table_hbm.at[idx_v.at[j]], rows_v.at[j], sem)
                for j in range(_CJ)
            ]
            for cp in copies:
                cp.wait()
            for j in range(_CJ):
                pb = (j % 2) * _JBS

                def row(r, acc):
                    for g in range(lanes):
                        sl = pl.ds(g * 16, 16)
                        rows_v[j, r, sl] = rows_v[j, r, sl] + pos_v[pb + r, sl]
                    return acc

                lax.fori_loop(0, _JBS, row, 0)
            pltpu.sync_copy(rows_v, out_hbm.at[pl.ds(jbase, _CJ)])
            return carry

        lax.fori_loop(0, n_chunks, chunk, 0)

    return emb


def kernel(tokens, token_table, pos_embedding):
    batch, seq = tokens.shape
    vocab, hidden = token_table.shape
    num_blocks = (batch * seq) // _JBS
    tok2 = tokens.reshape(num_blocks, _JBS).astype(jnp.int32)
    pos2 = pos_embedding.reshape(pos_embedding.shape[-2], hidden)
    out = _build(num_blocks, vocab, hidden, pos2.shape[0])(
        tok2, token_table, pos2
    )
    return out.reshape(batch, seq, hidden)


# speedup vs baseline: 1.1012x; 1.1012x over previous
import functools

import jax
import jax.numpy as jnp
from jax import lax
from jax.experimental import pallas as pl
from jax.experimental.pallas import tpu as pltpu
from jax.experimental.pallas import tpu_sc as plsc

_NC = 2
_NS = 16
_NW = _NC * _NS

_JBS = 100
_CJ = 8


@functools.cache
def _build(num_blocks, vocab, hidden, seq_len):
    del vocab
    blocks_per_w = num_blocks // _NW
    n_pairs = blocks_per_w // (2 * _CJ)
    lanes = hidden // 16
    half = seq_len // 2

    mesh = plsc.VectorSubcoreMesh(core_axis_name="c", subcore_axis_name="s")

    @functools.partial(
        pl.kernel,
        out_type=jax.ShapeDtypeStruct((num_blocks, _JBS, hidden), jnp.float32),
        mesh=mesh,
        compiler_params=pltpu.CompilerParams(use_tc_tiling_on_sc=False),
        scratch_types=[
            pltpu.VMEM((_CJ, _JBS), jnp.int32),
            pltpu.VMEM((_CJ, _JBS), jnp.int32),
            pltpu.VMEM((_CJ, _JBS, hidden), jnp.float32),
            pltpu.VMEM((_CJ, _JBS, hidden), jnp.float32),
            pltpu.VMEM((seq_len, hidden), jnp.float32),
            pltpu.SemaphoreType.DMA,
            pltpu.SemaphoreType.DMA,
        ],
    )
    def emb(tok_hbm, table_hbm, pos_hbm, out_hbm,
            idx0, idx1, rows0, rows1, pos_v, sg0, sg1):
        wid = lax.axis_index("s") * _NC + lax.axis_index("c")
        base = wid * blocks_per_w
        limit = num_blocks - _CJ
        pltpu.sync_copy(pos_hbm, pos_v)

        def fire(jb, idx_v, rows_v, sem):
            pltpu.sync_copy(tok_hbm.at[pl.ds(jb, _CJ)], idx_v)
            for j in range(_CJ):
                pltpu.async_copy(table_hbm.at[idx_v.at[j]], rows_v.at[j], sem)

        def drain(rows_v, sem):
            pltpu.make_async_copy(out_hbm.at[pl.ds(0, _CJ)], rows_v, sem).wait()

        def add_and_store(jb, rows_v):
            def row(r, acc):
                p0 = [pos_v[r, pl.ds(g * 16, 16)] for g in range(lanes)]
                p1 = [pos_v[half + r, pl.ds(g * 16, 16)] for g in range(lanes)]
                for j in range(_CJ):
                    pj = p1 if (j % 2) else p0
                    for g in range(lanes):
                        sl = pl.ds(g * 16, 16)
                        rows_v[j, r, sl] = rows_v[j, r, sl] + pj[g]
                return acc

            lax.fori_loop(0, _JBS, row, 0)
            pltpu.sync_copy(rows_v, out_hbm.at[pl.ds(jb, _CJ)])

        fire(base, idx0, rows0, sg0)

        def pair(p, carry):
            jb_a = base + (2 * p) * _CJ
            jb_b = jb_a + _CJ
            jb_c = jnp.minimum(jb_a + 2 * _CJ, limit)
            fire(jb_b, idx1, rows1, sg1)
            drain(rows0, sg0)
            add_and_store(jb_a, rows0)
            fire(jb_c, idx0, rows0, sg0)
            drain(rows1, sg1)
            add_and_store(jb_b, rows1)
            return carry

        lax.fori_loop(0, n_pairs, pair, 0)
        drain(rows0, sg0)

    return emb


def kernel(tokens, token_table, pos_embedding):
    batch, seq = tokens.shape
    vocab, hidden = token_table.shape
    num_blocks = (batch * seq) // _JBS
    tok2 = tokens.reshape(num_blocks, _JBS).astype(jnp.int32)
    pos2 = pos_embedding.reshape(pos_embedding.shape[-2], hidden)
    out = _build(num_blocks, vocab, hidden, pos2.shape[0])(
        tok2, token_table, pos2
    )
    return out.reshape(batch, seq, hidden)


# speedup vs baseline: 1.1028x; 1.0015x over previous
import functools

import jax
import jax.numpy as jnp
from jax import lax
from jax.experimental import pallas as pl
from jax.experimental.pallas import tpu as pltpu
from jax.experimental.pallas import tpu_sc as plsc

_NC = 2
_NS = 16
_NW = _NC * _NS

_CS = 4


@functools.cache
def _build(batch, seq, vocab, hidden):
    del vocab
    seqs_per_w = batch // _NW
    n_pairs = seqs_per_w // (2 * _CS)
    lanes = hidden // 16
    g0 = min(128, (seq // 2 + 7) // 8 * 8)
    splits = ((0, g0), (g0, seq - g0))

    mesh = plsc.VectorSubcoreMesh(core_axis_name="c", subcore_axis_name="s")

    @functools.partial(
        pl.kernel,
        out_type=jax.ShapeDtypeStruct((batch, seq, hidden), jnp.float32),
        mesh=mesh,
        compiler_params=pltpu.CompilerParams(use_tc_tiling_on_sc=False),
        scratch_types=[
            pltpu.VMEM((_CS, seq), jnp.int32),
            pltpu.VMEM((_CS, seq), jnp.int32),
            pltpu.VMEM((_CS, seq, hidden), jnp.float32),
            pltpu.VMEM((_CS, seq, hidden), jnp.float32),
            pltpu.VMEM((seq, hidden), jnp.float32),
            pltpu.SemaphoreType.DMA,
            pltpu.SemaphoreType.DMA,
        ],
    )
    def emb(tok_hbm, table_hbm, pos_hbm, out_hbm,
            idx0, idx1, rows0, rows1, pos_v, sg0, sg1):
        wid = lax.axis_index("s") * _NC + lax.axis_index("c")
        base = wid * seqs_per_w
        limit = batch - _CS
        pltpu.sync_copy(pos_hbm.at[0], pos_v)

        def fire(sb, idx_v, rows_v, sem):
            pltpu.sync_copy(tok_hbm.at[pl.ds(sb, _CS)], idx_v)
            for s in range(_CS):
                for off, ln in splits:
                    sl = pl.ds(off, ln)
                    pltpu.async_copy(
                        table_hbm.at[idx_v.at[s, sl]], rows_v.at[s, sl], sem
                    )

        def drain(rows_v, sem):
            pltpu.make_async_copy(out_hbm.at[pl.ds(0, _CS)], rows_v, sem).wait()

        def add_and_store(sb, rows_v):
            def row(r, acc):
                p = [pos_v[r, pl.ds(g * 16, 16)] for g in range(lanes)]
                for s in range(_CS):
                    for g in range(lanes):
                        sl = pl.ds(g * 16, 16)
                        rows_v[s, r, sl] = rows_v[s, r, sl] + p[g]
                return acc

            lax.fori_loop(0, seq, row, 0)
            pltpu.sync_copy(rows_v, out_hbm.at[pl.ds(sb, _CS)])

        fire(base, idx0, rows0, sg0)

        def pair(p, carry):
            sb_a = base + (2 * p) * _CS
            sb_b = sb_a + _CS
            sb_c = jnp.minimum(sb_a + 2 * _CS, limit)
            fire(sb_b, idx1, rows1, sg1)
            drain(rows0, sg0)
            add_and_store(sb_a, rows0)
            fire(sb_c, idx0, rows0, sg0)
            drain(rows1, sg1)
            add_and_store(sb_b, rows1)
            return carry

        lax.fori_loop(0, n_pairs, pair, 0)
        drain(rows0, sg0)

    return emb


def kernel(tokens, token_table, pos_embedding):
    batch, seq = tokens.shape
    vocab, hidden = token_table.shape
    return _build(batch, seq, vocab, hidden)(
        tokens.astype(jnp.int32), token_table, pos_embedding
    )
